# Initial kernel scaffold; baseline (speedup 1.0000x reference)
import jax
import jax.numpy as jnp
from jax.experimental import pallas as pl


def kernel(x, edge_index, W1, b1, W2, b2):
    raise NotImplementedError("write your pallas kernel here")


# speedup vs baseline: 8.1696x; 8.1696x over previous
import functools

import jax
import jax.numpy as jnp
from jax import lax
from jax.experimental import pallas as pl
from jax.experimental.pallas import tpu as pltpu
from jax.experimental.pallas import tpu_sc as plsc

CH = 128
BR = 512
N_TILES = 16
FH = 128


def _ceil_to(a, m):
    return (a + m - 1) // m * m


def _make_deg(npad, nch_tile):
    half = npad // 2
    rpt = npad // N_TILES
    hrpt = half // N_TILES
    mesh = plsc.VectorSubcoreMesh(core_axis_name="c", subcore_axis_name="s")

    @functools.partial(
        pl.kernel,
        out_type=jax.ShapeDtypeStruct((npad, 8), jnp.float32),
        mesh=mesh,
        scratch_types=[
            pltpu.VMEM((nch_tile, CH), jnp.int32),
            pltpu.VMEM((CH, 8), jnp.float32),
            pltpu.VMEM_SHARED((npad + 1, 8), jnp.float32),
        ],
    )
    def deg_kernel(dst2d, ones_hbm, deg_out, dst_v, ones_v, acc):
        c = lax.axis_index("c")
        s = lax.axis_index("s")
        pltpu.sync_copy(dst2d.at[pl.ds(s * nch_tile, nch_tile)], dst_v)
        pltpu.sync_copy(ones_hbm.at[pl.ds(0, CH)], ones_v)
        pltpu.sync_copy(ones_hbm.at[pl.ds(s * rpt, rpt)],
                        acc.at[pl.ds(s * rpt, rpt)])
        plsc.subcore_barrier()

        def body(j, carry):
            pltpu.sync_copy(ones_v, acc.at[dst_v.at[j]], add=True)
            return carry

        lax.fori_loop(0, nch_tile, body, 0)
        plsc.subcore_barrier()
        r0 = c * half + s * hrpt
        pltpu.sync_copy(acc.at[pl.ds(r0, hrpt)], deg_out.at[pl.ds(r0, hrpt)])

    return deg_kernel


def _make_prop(npad, nch_tile):
    rpt = npad // N_TILES
    mesh = plsc.VectorSubcoreMesh(core_axis_name="c", subcore_axis_name="s")

    @functools.partial(
        pl.kernel,
        out_type=[jax.ShapeDtypeStruct((npad, FH), jnp.float32)] * 2,
        mesh=mesh,
        scratch_types=[
            pltpu.VMEM((nch_tile, CH), jnp.int32),
            pltpu.VMEM((nch_tile, CH), jnp.int32),
            pltpu.VMEM((CH, FH), jnp.float32),
            pltpu.VMEM_SHARED((npad + 1, FH), jnp.float32),
            pltpu.SemaphoreType.DMA,
        ],
    )
    def prop_kernel(y_lo, y_hi, src2d, dst2d, s_lo, s_hi,
                    src_v, dst_v, buf, acc, sem):
        c = lax.axis_index("c")
        s = lax.axis_index("s")
        pltpu.sync_copy(src2d.at[pl.ds(s * nch_tile, nch_tile)], src_v)
        pltpu.sync_copy(dst2d.at[pl.ds(s * nch_tile, nch_tile)], dst_v)
        r0 = s * rpt

        def run(y, out):
            pltpu.sync_copy(y.at[pl.ds(r0, rpt)], acc.at[pl.ds(r0, rpt)])
            plsc.subcore_barrier()

            def body(j, carry):
                pltpu.async_copy(y.at[src_v.at[j]], buf, sem).wait()
                pltpu.sync_copy(buf, acc.at[dst_v.at[j]], add=True)
                return carry

            lax.fori_loop(0, nch_tile, body, 0)
            plsc.subcore_barrier()
            pltpu.sync_copy(acc.at[pl.ds(r0, rpt)], out.at[pl.ds(r0, rpt)])

        @pl.when(c == 0)
        def _():
            run(y_lo, s_lo)

        @pl.when(c == 1)
        def _():
            run(y_hi, s_hi)

    return prop_kernel


def _make_scale(npad):
    def body(deg_ref, x_ref, lo_ref, hi_ref):
        dinv = lax.rsqrt(deg_ref[:, 0:1])
        t = x_ref[...] * dinv
        lo_ref[...] = t[:, :FH]
        hi_ref[...] = t[:, FH:]

    return pl.pallas_call(
        body,
        grid=(npad // BR,),
        in_specs=[pl.BlockSpec((BR, 8), lambda i: (i, 0)),
                  pl.BlockSpec((BR, 2 * FH), lambda i: (i, 0))],
        out_specs=[pl.BlockSpec((BR, FH), lambda i: (i, 0)),
                   pl.BlockSpec((BR, FH), lambda i: (i, 0))],
        out_shape=[jax.ShapeDtypeStruct((npad, FH), jnp.float32)] * 2,
    )


def _make_layer(npad, fout, post_scale):
    def body(deg_ref, lo_ref, hi_ref, w_ref, b_ref, *outs):
        dinv = lax.rsqrt(deg_ref[:, 0:1])
        sfull = jnp.concatenate([lo_ref[...], hi_ref[...]], axis=1) * dinv
        h = jnp.dot(sfull, w_ref[...], preferred_element_type=jnp.float32)
        h = jnp.maximum(h + b_ref[...], 0.0)
        if post_scale:
            h = h * dinv
            outs[0][...] = h[:, :FH]
            outs[1][...] = h[:, FH:]
        else:
            outs[0][...] = h

    if post_scale:
        out_specs = [pl.BlockSpec((BR, FH), lambda i: (i, 0)),
                     pl.BlockSpec((BR, FH), lambda i: (i, 0))]
        out_shape = [jax.ShapeDtypeStruct((npad, FH), jnp.float32)] * 2
    else:
        out_specs = pl.BlockSpec((BR, fout), lambda i: (i, 0))
        out_shape = jax.ShapeDtypeStruct((npad, fout), jnp.float32)

    return pl.pallas_call(
        body,
        grid=(npad // BR,),
        in_specs=[pl.BlockSpec((BR, 8), lambda i: (i, 0)),
                  pl.BlockSpec((BR, FH), lambda i: (i, 0)),
                  pl.BlockSpec((BR, FH), lambda i: (i, 0)),
                  pl.BlockSpec((2 * FH, fout), lambda i: (0, 0)),
                  pl.BlockSpec((1, fout), lambda i: (0, 0))],
        out_specs=out_specs,
        out_shape=out_shape,
    )


def kernel(x, edge_index, W1, b1, W2, b2):
    n, f = x.shape
    e = edge_index.shape[1]
    npad = _ceil_to(n, 1024)
    epad = _ceil_to(e, CH * N_TILES * 8)
    nch_tile = epad // (CH * N_TILES)

    ei = edge_index.astype(jnp.int32)
    src = jnp.pad(ei[0], (0, epad - e))
    dst = jnp.pad(ei[1], (0, epad - e), constant_values=npad)
    src2d = src.reshape(-1, CH)
    dst2d = dst.reshape(-1, CH)
    x_p = jnp.pad(x, ((0, npad - n), (0, 0)))
    ones = jnp.ones((npad, 8), jnp.float32)

    deg_k = _make_deg(npad, nch_tile)
    prop_k = _make_prop(npad, nch_tile)
    scale_k = _make_scale(npad)
    layer1_k = _make_layer(npad, f, post_scale=True)
    layer2_k = _make_layer(npad, W2.shape[1], post_scale=False)

    deg = deg_k(dst2d, ones)
    y_lo, y_hi = scale_k(deg, x_p)
    s_lo, s_hi = prop_k(y_lo, y_hi, src2d, dst2d)
    y2_lo, y2_hi = layer1_k(deg, s_lo, s_hi, W1, b1.reshape(1, -1))
    t_lo, t_hi = prop_k(y2_lo, y2_hi, src2d, dst2d)
    out = layer2_k(deg, t_lo, t_hi, W2, b2.reshape(1, -1))
    return out[:n]
